# baseline (speedup 1.0000x reference)
import functools

import jax
import jax.numpy as jnp
from jax import lax
from jax.experimental import pallas as pl
from jax.experimental.pallas import tpu as pltpu
from jax.experimental.pallas import tpu_sc as plsc

K = 5
NUM_ROWS = 128
NUM_COLS = 32768
NC = 2
NS = 16
NW = NC * NS
RPW = NUM_ROWS // NW
L = 16
CH = 4096
NCH = NUM_COLS // CH
VPC = CH // L

_MESH = plsc.VectorSubcoreMesh(
    core_axis_name="c", subcore_axis_name="s", num_cores=NC, num_subcores=NS)


def _sc_body(xflat_hbm, label_hbm, out_hbm, lab_v, v_v, buf_v, flags_v, sem):
    wid = lax.axis_index("s") * NC + lax.axis_index("c")
    lane = lax.iota(jnp.int32, L)
    rows16 = jnp.minimum(wid * RPW + lane, NUM_ROWS - 1)
    pltpu.async_copy(label_hbm.at[rows16], lab_v, sem).wait()
    lab16 = lab_v[...]
    idx16 = rows16 * NUM_COLS + lab16
    pltpu.async_copy(xflat_hbm.at[idx16], v_v, sem).wait()
    v16 = v_v[...]

    flags16 = jnp.zeros((L,), jnp.float32)
    neg_inf = jnp.float32(-jnp.inf)

    for r in range(RPW):
        sel = lane == r
        v_r = jnp.max(jnp.where(sel, v16, neg_inf))
        l_r = jnp.max(jnp.where(sel, lab16, -1))
        vsplat = jnp.full((L,), v_r)
        row_base = (wid * RPW + r) * NUM_COLS

        def chunk_scan(k, acc, _buf=buf_v):
            x16 = _buf[pl.ds(k * L, L)]
            return acc + jnp.where(
                x16 > vsplat, 65536, jnp.where(x16 == vsplat, 1, 0))

        def cond(carry):
            c, acc = carry
            packed = jnp.sum(acc)
            return jnp.logical_and(c < NCH, (packed >> 16) < K)

        def body(carry, _vs=vsplat, _rb=row_base):
            c, acc = carry
            pltpu.sync_copy(xflat_hbm.at[pl.ds(_rb + c * CH, CH)], buf_v)
            acc = lax.fori_loop(0, VPC, chunk_scan, acc, unroll=8)
            return c + 1, acc

        c_fin, acc = lax.while_loop(
            cond, body, (jnp.int32(0), jnp.zeros((L,), jnp.int32)))
        packed = jnp.sum(acc)
        gt = packed >> 16
        eq = packed & 0xFFFF

        def tie_pass(_vs=vsplat, _rb=row_base, _l=l_r):
            def tbody(c, cnt):
                pltpu.sync_copy(xflat_hbm.at[pl.ds(_rb + c * CH, CH)], buf_v)

                def tscan(k, cnt2, _c=c):
                    x16 = buf_v[pl.ds(k * L, L)]
                    pos = _c * CH + k * L + lane
                    hit = jnp.logical_and(x16 == _vs, pos < _l)
                    return cnt2 + jnp.where(hit, 1, 0)

                return lax.fori_loop(0, VPC, tscan, cnt)

            cnt = lax.fori_loop(0, NCH, tbody, jnp.zeros((L,), jnp.int32))
            return jnp.sum(cnt)

        need_ties = jnp.logical_and(eq != 1, gt < K)
        eq_before = lax.cond(need_ties, tie_pass, lambda: jnp.int32(0))
        correct = ((gt + eq_before) < K).astype(jnp.float32)
        flags16 = jnp.where(sel, correct, flags16)

    flags_v[...] = flags16
    pltpu.sync_copy(flags_v, out_hbm.at[wid])


_sc_count = functools.partial(
    pl.kernel,
    out_type=jax.ShapeDtypeStruct((NW, L), jnp.float32),
    mesh=_MESH,
    compiler_params=pltpu.CompilerParams(needs_layout_passes=False),
    scratch_types=[
        pltpu.VMEM((L,), jnp.int32),
        pltpu.VMEM((L,), jnp.float32),
        pltpu.VMEM((CH,), jnp.float32),
        pltpu.VMEM((L,), jnp.float32),
        pltpu.SemaphoreType.DMA,
    ],
)(_sc_body)


def _mean_body(flags_ref, out_ref):
    out_ref[...] = (jnp.sum(flags_ref[...]) * (1.0 / NUM_ROWS)).reshape(1, 1)


def kernel(output, label):
    flags = _sc_count(output.reshape(-1), label)
    acc = pl.pallas_call(
        _mean_body,
        out_shape=jax.ShapeDtypeStruct((1, 1), jnp.float32),
    )(flags)
    return acc[0, 0]


# speedup vs baseline: 1.5097x; 1.5097x over previous
import functools

import jax
import jax.numpy as jnp
from jax import lax
from jax.experimental import pallas as pl
from jax.experimental.pallas import tpu as pltpu
from jax.experimental.pallas import tpu_sc as plsc

K = 5
NUM_ROWS = 128
NUM_COLS = 32768
NC = 2
NS = 16
NW = NC * NS
RPW = NUM_ROWS // NW
L = 16
CH = 4096
NCH = NUM_COLS // CH
VPC = CH // L

_MESH = plsc.VectorSubcoreMesh(
    core_axis_name="c", subcore_axis_name="s", num_cores=NC, num_subcores=NS)


def _sc_body(x_hbm, label_hbm, out_hbm, lab_v, v_v, buf_v, flags_v, sem):
    wid = lax.axis_index("s") * NC + lax.axis_index("c")
    lane = lax.iota(jnp.int32, L)
    rows16 = jnp.minimum(wid * RPW + lane, NUM_ROWS - 1)
    pltpu.async_copy(label_hbm.at[rows16], lab_v, sem).wait()
    lab16 = lab_v[...]

    flags16 = jnp.zeros((L,), jnp.float32)
    neg_inf = jnp.float32(-jnp.inf)

    for r in range(RPW):
        sel = lane == r
        l_r = jnp.max(jnp.where(sel, lab16, -1))
        row = wid * RPW + r
        c0 = (l_r // L) * L
        pltpu.sync_copy(x_hbm.at[row, pl.ds(c0, L)], v_v)
        v_r = jnp.max(jnp.where(lane == l_r - c0, v_v[...], neg_inf))
        vsplat = jnp.full((L,), v_r)

        def chunk_scan(k, acc, _buf=buf_v, _vs=vsplat):
            x16 = _buf[pl.ds(k * L, L)]
            return acc + jnp.where(
                x16 > _vs, 65536, jnp.where(x16 == _vs, 1, 0))

        def cond(carry):
            c, acc = carry
            packed = jnp.sum(acc)
            return jnp.logical_and(c < NCH, (packed >> 16) < K)

        def body(carry, _row=row, _scan=chunk_scan):
            c, acc = carry
            pltpu.sync_copy(x_hbm.at[_row, pl.ds(c * CH, CH)], buf_v)
            acc = lax.fori_loop(0, VPC, _scan, acc, unroll=8)
            return c + 1, acc

        c_fin, acc = lax.while_loop(
            cond, body, (jnp.int32(0), jnp.zeros((L,), jnp.int32)))
        packed = jnp.sum(acc)
        gt = packed >> 16
        eq = packed & 0xFFFF

        def tie_pass(_vs=vsplat, _row=row, _l=l_r):
            def tbody(c, cnt):
                pltpu.sync_copy(x_hbm.at[_row, pl.ds(c * CH, CH)], buf_v)

                def tscan(k, cnt2, _c=c):
                    x16 = buf_v[pl.ds(k * L, L)]
                    pos = _c * CH + k * L + lane
                    hit = jnp.logical_and(x16 == _vs, pos < _l)
                    return cnt2 + jnp.where(hit, 1, 0)

                return lax.fori_loop(0, VPC, tscan, cnt)

            cnt = lax.fori_loop(0, NCH, tbody, jnp.zeros((L,), jnp.int32))
            return jnp.sum(cnt)

        need_ties = jnp.logical_and(eq != 1, gt < K)
        eq_before = lax.cond(need_ties, tie_pass, lambda: jnp.int32(0))
        correct = ((gt + eq_before) < K).astype(jnp.float32)
        flags16 = jnp.where(sel, correct, flags16)

    flags_v[...] = flags16
    pltpu.sync_copy(flags_v, out_hbm.at[wid])


_sc_count = functools.partial(
    pl.kernel,
    out_type=jax.ShapeDtypeStruct((NW, L), jnp.float32),
    mesh=_MESH,
    compiler_params=pltpu.CompilerParams(needs_layout_passes=False),
    scratch_types=[
        pltpu.VMEM((L,), jnp.int32),
        pltpu.VMEM((L,), jnp.float32),
        pltpu.VMEM((CH,), jnp.float32),
        pltpu.VMEM((L,), jnp.float32),
        pltpu.SemaphoreType.DMA,
    ],
)(_sc_body)


def _mean_body(flags_ref, out_ref):
    out_ref[...] = (jnp.sum(flags_ref[...]) * (1.0 / NUM_ROWS)).reshape(1, 1)


def kernel(output, label):
    flags = _sc_count(output, label)
    acc = pl.pallas_call(
        _mean_body,
        out_shape=jax.ShapeDtypeStruct((1, 1), jnp.float32),
    )(flags)
    return acc[0, 0]
